# baseline (speedup 1.0000x reference)
import functools

import jax
import jax.numpy as jnp
from jax import lax
from jax.experimental import pallas as pl
from jax.experimental.pallas import tpu as pltpu
from jax.experimental.pallas import tpu_sc as plsc

EMBED = 64
B = 16384
L = 20
ROWS = B * L

NC = 2
NS = 16
NW = NC * NS
ROWS_PER_W = ROWS // NW
CHUNK = 128
NCHUNK = ROWS_PER_W // CHUNK

_mesh = plsc.VectorSubcoreMesh(core_axis_name="c", subcore_axis_name="s")


@functools.partial(
    pl.kernel,
    mesh=_mesh,
    out_type=jax.ShapeDtypeStruct((ROWS, EMBED), jnp.float32),
    scratch_types=[
        pltpu.VMEM((CHUNK,), jnp.int32),
        pltpu.VMEM((CHUNK, EMBED), jnp.float32),
        pltpu.SemaphoreType.DMA,
    ],
    compiler_params=pltpu.CompilerParams(use_tc_tiling_on_sc=False),
)
def _sc_gather(table_hbm, idx_hbm, out_hbm, idx_v, rows_v, sem):
    wid = lax.axis_index("s") * NC + lax.axis_index("c")
    base = wid * ROWS_PER_W

    def body(i, carry):
        off = pl.multiple_of(base + i * CHUNK, CHUNK)
        pltpu.sync_copy(idx_hbm.at[pl.ds(off, CHUNK)], idx_v)
        pltpu.async_copy(table_hbm.at[idx_v], rows_v, sem).wait()
        pltpu.sync_copy(rows_v, out_hbm.at[pl.ds(off, CHUNK)])
        return carry

    lax.fori_loop(0, NCHUNK, body, 0)


_MM_BLK = 4096


def _mm_body(x_ref, w_ref, b_ref, o_ref):
    x = x_ref[...]
    w = w_ref[...]
    acc = lax.dot_general(x, w, (((1,), (1,)), ((), ())),
                          preferred_element_type=jnp.float32)
    o_ref[...] = acc + b_ref[...]


_mm = pl.pallas_call(
    _mm_body,
    grid=(ROWS // _MM_BLK,),
    in_specs=[
        pl.BlockSpec((_MM_BLK, EMBED), lambda i: (i, 0)),
        pl.BlockSpec((EMBED, EMBED), lambda i: (0, 0)),
        pl.BlockSpec((1, EMBED), lambda i: (0, 0)),
    ],
    out_specs=pl.BlockSpec((_MM_BLK, EMBED), lambda i: (i, 0)),
    out_shape=jax.ShapeDtypeStruct((ROWS, EMBED), jnp.float32),
)


def kernel(input_ids, embedding, W, b):
    idx = input_ids.reshape(ROWS).astype(jnp.int32)
    gathered = _sc_gather(embedding, idx)
    out = _mm(gathered, W, b.reshape(1, EMBED))
    return out.reshape(B, L, EMBED)


# speedup vs baseline: 1.1322x; 1.1322x over previous
import functools

import jax
import jax.numpy as jnp
from jax import lax
from jax.experimental import pallas as pl
from jax.experimental.pallas import tpu as pltpu
from jax.experimental.pallas import tpu_sc as plsc

EMBED = 64
B = 16384
L = 20

NC = 2
NS = 16
NW = NC * NS
B_PER_W = B // NW
NB = 16
NCHUNK = B_PER_W // NB

_mesh = plsc.VectorSubcoreMesh(core_axis_name="c", subcore_axis_name="s")


@functools.partial(
    pl.kernel,
    mesh=_mesh,
    out_type=jax.ShapeDtypeStruct((B, L, EMBED), jnp.float32),
    scratch_types=[
        pltpu.VMEM((NB, L), jnp.int32),
        pltpu.VMEM((NB, L, EMBED), jnp.float32),
        pltpu.SemaphoreType.DMA,
    ],
    compiler_params=pltpu.CompilerParams(use_tc_tiling_on_sc=False),
)
def _sc_gather(table_hbm, ids_hbm, out_hbm, idx_v, rows_v, sem):
    wid = lax.axis_index("s") * NC + lax.axis_index("c")
    base = wid * B_PER_W

    def body(c, carry):
        b0 = base + c * NB
        pltpu.sync_copy(ids_hbm.at[pl.ds(b0, NB)], idx_v)
        copies = [
            pltpu.async_copy(table_hbm.at[idx_v.at[r]], rows_v.at[r], sem)
            for r in range(NB)
        ]
        for cp in copies:
            cp.wait()
        pltpu.sync_copy(rows_v, out_hbm.at[pl.ds(b0, NB)])
        return carry

    lax.fori_loop(0, NCHUNK, body, 0)


_BB = 256


def _mm_body(x_ref, w_ref, b_ref, o_ref):
    x = x_ref[...].reshape(_BB * L, EMBED)
    y = lax.dot_general(x, w_ref[...], (((1,), (1,)), ((), ())),
                        preferred_element_type=jnp.float32)
    o_ref[...] = y.reshape(_BB, L, EMBED) + b_ref[...]


_mm = pl.pallas_call(
    _mm_body,
    grid=(B // _BB,),
    in_specs=[
        pl.BlockSpec((_BB, L, EMBED), lambda i: (i, 0, 0)),
        pl.BlockSpec((EMBED, EMBED), lambda i: (0, 0)),
        pl.BlockSpec((1, 1, EMBED), lambda i: (0, 0, 0)),
    ],
    out_specs=pl.BlockSpec((_BB, L, EMBED), lambda i: (i, 0, 0)),
    out_shape=jax.ShapeDtypeStruct((B, L, EMBED), jnp.float32),
)


def kernel(input_ids, embedding, W, b):
    ids = input_ids.astype(jnp.int32)
    gathered = _sc_gather(embedding, ids)
    return _mm(gathered, W, b.reshape(1, 1, EMBED))
